# Initial kernel scaffold; baseline (speedup 1.0000x reference)
import jax
import jax.numpy as jnp
from jax.experimental import pallas as pl


def kernel(x, edge_index, edge_attr, W1, att1_src, att1_dst, b1, W2, att2_src, att2_dst, b2):
    raise NotImplementedError("write your pallas kernel here")


# speedup vs baseline: 13.5738x; 13.5738x over previous
import functools

import jax
import jax.numpy as jnp
from jax import lax
from jax.experimental import pallas as pl
from jax.experimental.pallas import tpu as pltpu
from jax.experimental.pallas import tpu_sc as plsc

HEADS = 8
N = 10000
NP = 10240
E = 320000
EP = 330240
K = 240
TE = EP // 16
NPASS = TE // K
SR = NP // 16
KA = 240
EA = EP // 32
NPASSA = EA // KA
QB = 80


RW = 128
CB = 64


def _project_body(x_ref, w_ref, as_ref, ad_ref, h_ref, asrc_ref, adst_ref,
                  *, C):
    HG = RW // C
    h = jnp.dot(x_ref[...], w_ref[0], preferred_element_type=jnp.float32)
    h_ref[0] = h
    for hg in range(HG):
        hs = h[:, hg * C:(hg + 1) * C]
        asrc_ref[0, hg] = (hs * as_ref[0, 0, hg * C:(hg + 1) * C]).sum(axis=-1)
        adst_ref[0, hg] = (hs * ad_ref[0, 0, hg * C:(hg + 1) * C]).sum(axis=-1)


def _project(xp, Wg, att_s, att_d, C, interpret=False):
    IN = xp.shape[1]
    G = Wg.shape[0]
    HG = RW // C
    BN = 512
    grid = (G, NP // BN)
    return pl.pallas_call(
        functools.partial(_project_body, C=C),
        grid=grid,
        in_specs=[
            pl.BlockSpec((BN, IN), lambda g, nb: (nb, 0)),
            pl.BlockSpec((1, IN, RW), lambda g, nb: (g, 0, 0)),
            pl.BlockSpec((1, 1, RW), lambda g, nb: (g, 0, 0)),
            pl.BlockSpec((1, 1, RW), lambda g, nb: (g, 0, 0)),
        ],
        out_specs=[
            pl.BlockSpec((1, BN, RW), lambda g, nb: (g, nb, 0)),
            pl.BlockSpec((1, HG, BN), lambda g, nb: (g, 0, nb)),
            pl.BlockSpec((1, HG, BN), lambda g, nb: (g, 0, nb)),
        ],
        out_shape=[
            jax.ShapeDtypeStruct((G, NP, RW), jnp.float32),
            jax.ShapeDtypeStruct((G, HG, NP), jnp.float32),
            jax.ShapeDtypeStruct((G, HG, NP), jnp.float32),
        ],
        interpret=interpret,
    )(xp, Wg, att_s, att_d)


def _epilogue_body(num_ref, den_ref, b_ref, o_ref, *, C, relu, bn):
    HG = RW // C
    G = num_ref.shape[0]
    v = num_ref[...]
    d = den_ref[0] + den_ref[1]
    m = jnp.zeros(v.shape[1:2] + (C,), jnp.float32)
    for hg in range(HG):
        m = m + (v[:, :, hg * C:(hg + 1) * C]
                 / d[:, hg, :, None]).sum(axis=0)
    m = m * (1.0 / HEADS) + b_ref[...][None, :]
    if relu:
        m = jnp.maximum(m, 0.0)
    nb = pl.program_id(0)
    row = nb * bn + lax.broadcasted_iota(jnp.int32, m.shape, 0)
    o_ref[...] = jnp.where(row < N, m, 0.0)


def _epilogue(num, den, b, relu, interpret=False):
    G = num.shape[0]
    HG = HEADS // G
    C = RW // HG
    BN = 512
    return pl.pallas_call(
        functools.partial(_epilogue_body, C=C, relu=relu, bn=BN),
        grid=(NP // BN,),
        in_specs=[
            pl.BlockSpec((G, BN, RW), lambda nb: (0, nb, 0)),
            pl.BlockSpec((2, G, HG, BN), lambda nb: (0, 0, 0, nb)),
            pl.BlockSpec((C,), lambda nb: (0,)),
        ],
        out_specs=pl.BlockSpec((BN, C), lambda nb: (nb, 0)),
        out_shape=jax.ShapeDtypeStruct((NP, C), jnp.float32),
        interpret=interpret,
    )(num, den, b)


def _sc_weights_body(asrc, adst, srcE, dstE, w2, denp,
                     s_buf, d_buf, w_buf, as_t, ad_t, den_sp):
    cid = lax.axis_index("c")
    sid = lax.axis_index("s")
    base = (cid * 16 + sid) * EA
    zeros = jnp.zeros((16,), jnp.float32)

    for head in range(HEADS):
        pltpu.sync_copy(asrc.at[pl.ds(head * NP, NP)], as_t)
        pltpu.sync_copy(adst.at[pl.ds(head * NP, NP)], ad_t)

        @pl.loop(0, KA // 16)
        def _(i):
            w_buf[pl.ds(i * 16, 16)] = zeros

        for r in range(SR // KA + (1 if SR % KA else 0)):
            n = min(KA, SR - r * KA)
            pltpu.sync_copy(w_buf.at[pl.ds(0, n)],
                            den_sp.at[pl.ds(sid * SR + r * KA, n)])
        plsc.subcore_barrier()

        @pl.loop(0, NPASSA)
        def _(j):
            off = base + j * KA
            pltpu.sync_copy(srcE.at[pl.ds(off, KA)], s_buf)
            for r in range(KA // QB):
                pltpu.sync_copy(dstE.at[pl.ds(off + r * QB, QB)], d_buf.at[r])

            @pl.loop(0, KA // 16)
            def _(i):
                s = s_buf[pl.ds(i * 16, 16)]
                d = d_buf[i * 16 // QB, pl.ds((i * 16) % QB, 16)]
                a = plsc.load_gather(as_t, [s]) + plsc.load_gather(ad_t, [d])
                w_buf[pl.ds(i * 16, 16)] = jnp.exp(jnp.maximum(a, a * 0.2))

            pltpu.sync_copy(w_buf, w2.at[pl.ds(head * EP + off, KA)])
            for r in range(KA // QB):
                pltpu.sync_copy(w_buf.at[pl.ds(r * QB, QB)],
                                den_sp.at[d_buf.at[r]], add=True)

        plsc.subcore_barrier()
        pltpu.sync_copy(
            den_sp.at[pl.ds(sid * SR, SR)],
            denp.at[pl.ds((cid * HEADS + head) * NP + sid * SR, SR)])
        plsc.subcore_barrier()


def _sc_weights(asrc, adst, srcE, dstE):
    mesh = plsc.VectorSubcoreMesh(core_axis_name="c", subcore_axis_name="s")
    f = pl.kernel(
        _sc_weights_body,
        out_type=[
            jax.ShapeDtypeStruct((HEADS * EP,), jnp.float32),
            jax.ShapeDtypeStruct((2 * HEADS * NP,), jnp.float32),
        ],
        mesh=mesh,
        compiler_params=pltpu.CompilerParams(needs_layout_passes=False),
        scratch_types=[
            pltpu.VMEM((KA,), jnp.int32),
            pltpu.VMEM((KA // QB, QB), jnp.int32),
            pltpu.VMEM((KA,), jnp.float32),
            pltpu.VMEM((NP,), jnp.float32),
            pltpu.VMEM((NP,), jnp.float32),
            pltpu.VMEM_SHARED((NP,), jnp.float32),
        ],
    )
    return f(asrc, adst, srcE, dstE)


def _sc_body(C, G, h2d, w2, srcE, dstE, num,
             src_buf, srh_buf, dst_buf, wa_buf, wb_buf, rows_buf,
             accum, sem):
    HG = RW // C
    cid = lax.axis_index("c")
    sid = lax.axis_index("s")
    base = sid * TE
    zeros = jnp.zeros((16,), jnp.float32)

    for gg in range(G // 2):
        g = cid * (G // 2) + gg
        ha = g * HG
        hb = g * HG + (HG - 1)

        @pl.loop(0, K)
        def _(e):
            for c in range(RW // 16):
                rows_buf[e, pl.ds(c * 16, 16)] = zeros

        for r in range(SR // K + (1 if SR % K else 0)):
            n = min(K, SR - r * K)
            pltpu.sync_copy(rows_buf.at[pl.ds(0, n)],
                            accum.at[pl.ds(sid * SR + r * K, n)])
        plsc.subcore_barrier()

        @pl.loop(0, NPASS)
        def _(j):
            off = base + j * K
            pltpu.sync_copy(srcE.at[pl.ds(off, K)], src_buf)
            for r in range(K // QB):
                pltpu.sync_copy(dstE.at[pl.ds(off + r * QB, QB)],
                                dst_buf.at[r])
            pltpu.sync_copy(w2.at[pl.ds(ha * EP + off, K)], wa_buf)
            if HG == 2:
                pltpu.sync_copy(w2.at[pl.ds(hb * EP + off, K)], wb_buf)

            @pl.loop(0, K // 16)
            def _(i):
                s = src_buf[pl.ds(i * 16, 16)]
                srh_buf[i * 16 // QB, pl.ds((i * 16) % QB, 16)] = s + g * NP

            cps = [pltpu.async_copy(h2d.at[srh_buf.at[r]],
                                    rows_buf.at[pl.ds(r * QB, QB)], sem)
                   for r in range(K // QB)]
            for cp in cps:
                cp.wait()

            @pl.loop(0, K)
            def _(e):
                idx = jnp.full((16,), e, jnp.int32)
                wa = plsc.load_gather(wa_buf, [idx])
                wb = plsc.load_gather(wb_buf, [idx]) if HG == 2 else wa
                for c in range(RW // 16):
                    w = wa if c < (RW // 32) else wb
                    rows_buf[e, pl.ds(c * 16, 16)] = (
                        rows_buf[e, pl.ds(c * 16, 16)] * w)

            for r in range(K // QB):
                pltpu.sync_copy(rows_buf.at[pl.ds(r * QB, QB)],
                                accum.at[dst_buf.at[r]], add=True)

        plsc.subcore_barrier()
        pltpu.sync_copy(accum.at[pl.ds(sid * SR, SR)],
                        num.at[g, pl.ds(sid * SR, SR)])
        plsc.subcore_barrier()


def _sc_aggregate(h2d, w2, srcE, dstE, C):
    G = h2d.shape[0] // NP
    mesh = plsc.VectorSubcoreMesh(core_axis_name="c", subcore_axis_name="s")
    f = pl.kernel(
        functools.partial(_sc_body, C, G),
        out_type=jax.ShapeDtypeStruct((G, NP, RW), jnp.float32),
        mesh=mesh,
        compiler_params=pltpu.CompilerParams(needs_layout_passes=False),
        scratch_types=[
            pltpu.VMEM((K,), jnp.int32),
            pltpu.VMEM((K // QB, QB), jnp.int32),
            pltpu.VMEM((K // QB, QB), jnp.int32),
            pltpu.VMEM((K,), jnp.float32),
            pltpu.VMEM((K,), jnp.float32),
            pltpu.VMEM((K, RW), jnp.float32),
            pltpu.VMEM_SHARED((NP, RW), jnp.float32),
            pltpu.SemaphoreType.DMA,
        ],
    )
    return f(h2d, w2, srcE, dstE)


def _layer(xp, W, att_s, att_d, b, srcE, dstE, C, relu):
    IN = xp.shape[1]
    G = HEADS * C // RW
    HG = RW // C
    Wg = W.reshape(IN, G, RW).transpose(1, 0, 2)
    h, asrc, adst = _project(xp, Wg, att_s.reshape(G, 1, RW),
                             att_d.reshape(G, 1, RW), C)
    w2, denp = _sc_weights(asrc.reshape(HEADS * NP), adst.reshape(HEADS * NP),
                           srcE, dstE)
    num = _sc_aggregate(h.reshape(G * NP, RW), w2, srcE, dstE, C)
    return _epilogue(num, denp.reshape(2, G, HG, NP), b, relu)


def kernel(x, edge_index, edge_attr, W1, att1_src, att1_dst, b1,
           W2, att2_src, att2_dst, b2):
    del edge_attr
    xp = jnp.pad(x, ((0, NP - N), (0, 0)))
    loops = jnp.arange(N, dtype=jnp.int32)
    pad = EP - E - N
    padv = N + (jnp.arange(pad, dtype=jnp.int32) % (NP - N))
    srcE = jnp.concatenate([edge_index[0], loops, padv])
    dstE = jnp.concatenate([edge_index[1], loops, padv])

    h1 = _layer(xp, W1, att1_src, att1_dst, b1, srcE, dstE, 128, True)
    out = _layer(h1, W2, att2_src, att2_dst, b2, srcE, dstE, 64, False)
    return out[:N]


# speedup vs baseline: 14.8882x; 1.0968x over previous
import functools

import jax
import jax.numpy as jnp
from jax import lax
from jax.experimental import pallas as pl
from jax.experimental.pallas import tpu as pltpu
from jax.experimental.pallas import tpu_sc as plsc

HEADS = 8
N = 10000
NP = 10240
E = 320000
EP = 332800
K = 160
TE = EP // 16
NPASS = TE // K
SR = NP // 16
KA = 160
EA = EP // 32
NPASSA = EA // KA
QB = 80


RW = 128
CB = 64


def _project_body(x_ref, w_ref, as_ref, ad_ref, h_ref, asrc_ref, adst_ref,
                  *, C):
    HG = RW // C
    h = jnp.dot(x_ref[...], w_ref[0], preferred_element_type=jnp.float32)
    h_ref[0] = h
    for hg in range(HG):
        hs = h[:, hg * C:(hg + 1) * C]
        asrc_ref[0, hg] = (hs * as_ref[0, 0, hg * C:(hg + 1) * C]).sum(axis=-1)
        adst_ref[0, hg] = (hs * ad_ref[0, 0, hg * C:(hg + 1) * C]).sum(axis=-1)


def _project(xp, Wg, att_s, att_d, C, interpret=False):
    IN = xp.shape[1]
    G = Wg.shape[0]
    HG = RW // C
    BN = 512
    grid = (G, NP // BN)
    return pl.pallas_call(
        functools.partial(_project_body, C=C),
        grid=grid,
        in_specs=[
            pl.BlockSpec((BN, IN), lambda g, nb: (nb, 0)),
            pl.BlockSpec((1, IN, RW), lambda g, nb: (g, 0, 0)),
            pl.BlockSpec((1, 1, RW), lambda g, nb: (g, 0, 0)),
            pl.BlockSpec((1, 1, RW), lambda g, nb: (g, 0, 0)),
        ],
        out_specs=[
            pl.BlockSpec((1, BN, RW), lambda g, nb: (g, nb, 0)),
            pl.BlockSpec((1, HG, BN), lambda g, nb: (g, 0, nb)),
            pl.BlockSpec((1, HG, BN), lambda g, nb: (g, 0, nb)),
        ],
        out_shape=[
            jax.ShapeDtypeStruct((G, NP, RW), jnp.float32),
            jax.ShapeDtypeStruct((G, HG, NP), jnp.float32),
            jax.ShapeDtypeStruct((G, HG, NP), jnp.float32),
        ],
        interpret=interpret,
    )(xp, Wg, att_s, att_d)


def _epilogue_body(num_ref, den_ref, b_ref, o_ref, *, C, relu, bn):
    HG = RW // C
    G = num_ref.shape[0]
    v = num_ref[...]
    d = den_ref[0] + den_ref[1]
    m = jnp.zeros(v.shape[1:2] + (C,), jnp.float32)
    for hg in range(HG):
        m = m + (v[:, :, hg * C:(hg + 1) * C]
                 / d[:, hg, :, None]).sum(axis=0)
    m = m * (1.0 / HEADS) + b_ref[...][None, :]
    if relu:
        m = jnp.maximum(m, 0.0)
    nb = pl.program_id(0)
    row = nb * bn + lax.broadcasted_iota(jnp.int32, m.shape, 0)
    o_ref[...] = jnp.where(row < N, m, 0.0)


def _epilogue(num, den, b, relu, interpret=False):
    G = num.shape[0]
    HG = HEADS // G
    C = RW // HG
    BN = 512
    return pl.pallas_call(
        functools.partial(_epilogue_body, C=C, relu=relu, bn=BN),
        grid=(NP // BN,),
        in_specs=[
            pl.BlockSpec((G, BN, RW), lambda nb: (0, nb, 0)),
            pl.BlockSpec((2, G, HG, BN), lambda nb: (0, 0, 0, nb)),
            pl.BlockSpec((C,), lambda nb: (0,)),
        ],
        out_specs=pl.BlockSpec((BN, C), lambda nb: (nb, 0)),
        out_shape=jax.ShapeDtypeStruct((NP, C), jnp.float32),
        interpret=interpret,
    )(num, den, b)


def _sc_weights_body(asrc, adst, srcE, dstE, w2, denp,
                     s_buf, d_buf, w_buf, as_t, ad_t, den_sp):
    cid = lax.axis_index("c")
    sid = lax.axis_index("s")
    base = (cid * 16 + sid) * EA
    zeros = jnp.zeros((16,), jnp.float32)

    for head in range(HEADS):
        pltpu.sync_copy(asrc.at[pl.ds(head * NP, NP)], as_t)
        pltpu.sync_copy(adst.at[pl.ds(head * NP, NP)], ad_t)

        @pl.loop(0, KA // 16)
        def _(i):
            w_buf[pl.ds(i * 16, 16)] = zeros

        for r in range(SR // KA + (1 if SR % KA else 0)):
            n = min(KA, SR - r * KA)
            pltpu.sync_copy(w_buf.at[pl.ds(0, n)],
                            den_sp.at[pl.ds(sid * SR + r * KA, n)])
        plsc.subcore_barrier()

        @pl.loop(0, NPASSA)
        def _(j):
            off = base + j * KA
            pltpu.sync_copy(srcE.at[pl.ds(off, KA)], s_buf)
            for r in range(KA // QB):
                pltpu.sync_copy(dstE.at[pl.ds(off + r * QB, QB)], d_buf.at[r])

            @pl.loop(0, KA // 16)
            def _(i):
                s = s_buf[pl.ds(i * 16, 16)]
                d = d_buf[i * 16 // QB, pl.ds((i * 16) % QB, 16)]
                a = plsc.load_gather(as_t, [s]) + plsc.load_gather(ad_t, [d])
                w_buf[pl.ds(i * 16, 16)] = jnp.exp(jnp.maximum(a, a * 0.2))

            pltpu.sync_copy(w_buf, w2.at[pl.ds(head * EP + off, KA)])
            for r in range(KA // QB):
                pltpu.sync_copy(w_buf.at[pl.ds(r * QB, QB)],
                                den_sp.at[d_buf.at[r]], add=True)

        plsc.subcore_barrier()
        pltpu.sync_copy(
            den_sp.at[pl.ds(sid * SR, SR)],
            denp.at[pl.ds((cid * HEADS + head) * NP + sid * SR, SR)])
        plsc.subcore_barrier()


def _sc_weights(asrc, adst, srcE, dstE):
    mesh = plsc.VectorSubcoreMesh(core_axis_name="c", subcore_axis_name="s")
    f = pl.kernel(
        _sc_weights_body,
        out_type=[
            jax.ShapeDtypeStruct((HEADS * EP,), jnp.float32),
            jax.ShapeDtypeStruct((2 * HEADS * NP,), jnp.float32),
        ],
        mesh=mesh,
        compiler_params=pltpu.CompilerParams(needs_layout_passes=False),
        scratch_types=[
            pltpu.VMEM((KA,), jnp.int32),
            pltpu.VMEM((KA // QB, QB), jnp.int32),
            pltpu.VMEM((KA,), jnp.float32),
            pltpu.VMEM((NP,), jnp.float32),
            pltpu.VMEM((NP,), jnp.float32),
            pltpu.VMEM_SHARED((NP,), jnp.float32),
        ],
    )
    return f(asrc, adst, srcE, dstE)


def _sc_body(C, G, h2d, w2, srcE, dstE, num, bufs, accum):
    HG = RW // C
    cid = lax.axis_index("c")
    sid = lax.axis_index("s")
    base = sid * TE
    zeros = jnp.zeros((16,), jnp.float32)

    def fetch(b, g, ha, hb, off):
        src_buf, srh_buf, dst_buf, wa_buf, wb_buf, rows_buf, sem = b
        pltpu.sync_copy(srcE.at[pl.ds(off, K)], src_buf)
        for r in range(K // QB):
            pltpu.sync_copy(dstE.at[pl.ds(off + r * QB, QB)], dst_buf.at[r])
        pltpu.sync_copy(w2.at[pl.ds(ha * EP + off, K)], wa_buf)
        if HG == 2:
            pltpu.sync_copy(w2.at[pl.ds(hb * EP + off, K)], wb_buf)

        @pl.loop(0, K // 16)
        def _(i):
            srh_buf[i * 16 // QB, pl.ds((i * 16) % QB, 16)] = (
                src_buf[pl.ds(i * 16, 16)] + g * NP)

        return [pltpu.async_copy(h2d.at[srh_buf.at[r]],
                                 rows_buf.at[pl.ds(r * QB, QB)], sem)
                for r in range(K // QB)]

    def process(b, cps):
        src_buf, srh_buf, dst_buf, wa_buf, wb_buf, rows_buf, sem = b
        for cp in cps:
            cp.wait()

        @pl.loop(0, K)
        def _(e):
            idx = jnp.full((16,), e, jnp.int32)
            wa = plsc.load_gather(wa_buf, [idx])
            wb = plsc.load_gather(wb_buf, [idx]) if HG == 2 else wa
            for c in range(RW // 16):
                w = wa if c < (RW // 32) else wb
                rows_buf[e, pl.ds(c * 16, 16)] = (
                    rows_buf[e, pl.ds(c * 16, 16)] * w)

        for r in range(K // QB):
            pltpu.sync_copy(rows_buf.at[pl.ds(r * QB, QB)],
                            accum.at[dst_buf.at[r]], add=True)

    for gg in range(G // 2):
        g = cid * (G // 2) + gg
        ha = g * HG
        hb = g * HG + (HG - 1)
        ba, bb = bufs
        rows_a = ba[5]

        @pl.loop(0, K)
        def _(e):
            for c in range(RW // 16):
                rows_a[e, pl.ds(c * 16, 16)] = zeros

        for r in range(SR // K + (1 if SR % K else 0)):
            n = min(K, SR - r * K)
            pltpu.sync_copy(rows_a.at[pl.ds(0, n)],
                            accum.at[pl.ds(sid * SR + r * K, n)])
        plsc.subcore_barrier()

        cp_a = fetch(ba, g, ha, hb, base)

        @pl.loop(0, NPASS // 2 - 1)
        def _(j):
            off = base + j * (2 * K)
            cp_b = fetch(bb, g, ha, hb, off + K)
            process(ba, cp_a)
            fetch(ba, g, ha, hb, off + 2 * K)
            process(bb, cp_b)

        off = base + (NPASS // 2 - 1) * (2 * K)
        cp_b = fetch(bb, g, ha, hb, off + K)
        process(ba, cp_a)
        process(bb, cp_b)

        plsc.subcore_barrier()
        pltpu.sync_copy(accum.at[pl.ds(sid * SR, SR)],
                        num.at[g, pl.ds(sid * SR, SR)])
        plsc.subcore_barrier()


def _sc_aggregate(h2d, w2, srcE, dstE, C):
    G = h2d.shape[0] // NP
    mesh = plsc.VectorSubcoreMesh(core_axis_name="c", subcore_axis_name="s")
    bufset = [
        pltpu.VMEM((K,), jnp.int32),
        pltpu.VMEM((K // QB, QB), jnp.int32),
        pltpu.VMEM((K // QB, QB), jnp.int32),
        pltpu.VMEM((K,), jnp.float32),
        pltpu.VMEM((K,), jnp.float32),
        pltpu.VMEM((K, RW), jnp.float32),
        pltpu.SemaphoreType.DMA,
    ]
    f = pl.kernel(
        functools.partial(_sc_body, C, G),
        out_type=jax.ShapeDtypeStruct((G, NP, RW), jnp.float32),
        mesh=mesh,
        compiler_params=pltpu.CompilerParams(needs_layout_passes=False),
        scratch_types=[
            (tuple(bufset), tuple(bufset)),
            pltpu.VMEM_SHARED((NP, RW), jnp.float32),
        ],
    )
    return f(h2d, w2, srcE, dstE)


def _layer(xp, W, att_s, att_d, b, srcE, dstE, C, relu):
    IN = xp.shape[1]
    G = HEADS * C // RW
    HG = RW // C
    Wg = W.reshape(IN, G, RW).transpose(1, 0, 2)
    h, asrc, adst = _project(xp, Wg, att_s.reshape(G, 1, RW),
                             att_d.reshape(G, 1, RW), C)
    w2, denp = _sc_weights(asrc.reshape(HEADS * NP), adst.reshape(HEADS * NP),
                           srcE, dstE)
    num = _sc_aggregate(h.reshape(G * NP, RW), w2, srcE, dstE, C)
    return _epilogue(num, denp.reshape(2, G, HG, NP), b, relu)


def kernel(x, edge_index, edge_attr, W1, att1_src, att1_dst, b1,
           W2, att2_src, att2_dst, b2):
    del edge_attr
    xp = jnp.pad(x, ((0, NP - N), (0, 0)))
    loops = jnp.arange(N, dtype=jnp.int32)
    pad = EP - E - N
    padv = N + (jnp.arange(pad, dtype=jnp.int32) % (NP - N))
    srcE = jnp.concatenate([edge_index[0], loops, padv])
    dstE = jnp.concatenate([edge_index[1], loops, padv])

    h1 = _layer(xp, W1, att1_src, att1_dst, b1, srcE, dstE, 128, True)
    out = _layer(h1, W2, att2_src, att2_dst, b2, srcE, dstE, 64, False)
    return out[:N]


# speedup vs baseline: 16.6881x; 1.1209x over previous
import functools

import jax
import jax.numpy as jnp
from jax import lax
from jax.experimental import pallas as pl
from jax.experimental.pallas import tpu as pltpu
from jax.experimental.pallas import tpu_sc as plsc

HEADS = 8
N = 10000
NP = 10240
E = 320000
EP = 332800
K = 160
TE = EP // 16
NPASS = TE // K
SR = NP // 16
KA = 160
EA = EP // 32
NPASSA = EA // KA
QB = 80


RW = 128
CB = 64


def _project_body(x_ref, w_ref, as_ref, ad_ref, h_ref, asrc_ref, adst_ref,
                  *, C):
    HG = RW // C
    h = jnp.dot(x_ref[...], w_ref[0], preferred_element_type=jnp.float32)
    h_ref[0] = h
    for hg in range(HG):
        hs = h[:, hg * C:(hg + 1) * C]
        asrc_ref[0, hg] = (hs * as_ref[0, 0, hg * C:(hg + 1) * C]).sum(axis=-1)
        adst_ref[0, hg] = (hs * ad_ref[0, 0, hg * C:(hg + 1) * C]).sum(axis=-1)


def _project(xp, Wg, att_s, att_d, C, interpret=False):
    IN = xp.shape[1]
    G = Wg.shape[0]
    HG = RW // C
    BN = 512
    grid = (G, NP // BN)
    return pl.pallas_call(
        functools.partial(_project_body, C=C),
        grid=grid,
        in_specs=[
            pl.BlockSpec((BN, IN), lambda g, nb: (nb, 0)),
            pl.BlockSpec((1, IN, RW), lambda g, nb: (g, 0, 0)),
            pl.BlockSpec((1, 1, RW), lambda g, nb: (g, 0, 0)),
            pl.BlockSpec((1, 1, RW), lambda g, nb: (g, 0, 0)),
        ],
        out_specs=[
            pl.BlockSpec((1, BN, RW), lambda g, nb: (g, nb, 0)),
            pl.BlockSpec((1, HG, BN), lambda g, nb: (g, 0, nb)),
            pl.BlockSpec((1, HG, BN), lambda g, nb: (g, 0, nb)),
        ],
        out_shape=[
            jax.ShapeDtypeStruct((G, NP, RW), jnp.float32),
            jax.ShapeDtypeStruct((G, HG, NP), jnp.float32),
            jax.ShapeDtypeStruct((G, HG, NP), jnp.float32),
        ],
        interpret=interpret,
    )(xp, Wg, att_s, att_d)


def _epilogue_body(num_ref, den_ref, b_ref, o_ref, *, C, relu, bn):
    HG = RW // C
    G = num_ref.shape[0]
    v = num_ref[...]
    d = den_ref[0] + den_ref[1]
    m = jnp.zeros(v.shape[1:2] + (C,), jnp.float32)
    for hg in range(HG):
        m = m + (v[:, :, hg * C:(hg + 1) * C]
                 / d[:, hg, :, None]).sum(axis=0)
    m = m * (1.0 / HEADS) + b_ref[...][None, :]
    if relu:
        m = jnp.maximum(m, 0.0)
    nb = pl.program_id(0)
    row = nb * bn + lax.broadcasted_iota(jnp.int32, m.shape, 0)
    o_ref[...] = jnp.where(row < N, m, 0.0)


def _epilogue(num, den, b, relu, interpret=False):
    G = num.shape[0]
    HG = HEADS // G
    C = RW // HG
    BN = 512
    return pl.pallas_call(
        functools.partial(_epilogue_body, C=C, relu=relu, bn=BN),
        grid=(NP // BN,),
        in_specs=[
            pl.BlockSpec((G, BN, RW), lambda nb: (0, nb, 0)),
            pl.BlockSpec((2, G, HG, BN), lambda nb: (0, 0, 0, nb)),
            pl.BlockSpec((C,), lambda nb: (0,)),
        ],
        out_specs=pl.BlockSpec((BN, C), lambda nb: (nb, 0)),
        out_shape=jax.ShapeDtypeStruct((NP, C), jnp.float32),
        interpret=interpret,
    )(num, den, b)


def _sc_weights_body(asrc, adst, srcE, dstE, w2, denp,
                     s_buf, d_buf, w_buf, as_t, ad_t, den_sp):
    cid = lax.axis_index("c")
    sid = lax.axis_index("s")
    base = (cid * 16 + sid) * EA
    zeros = jnp.zeros((16,), jnp.float32)

    for head in range(HEADS):
        pltpu.sync_copy(asrc.at[pl.ds(head * NP, NP)], as_t)
        pltpu.sync_copy(adst.at[pl.ds(head * NP, NP)], ad_t)

        @pl.loop(0, KA // 16)
        def _(i):
            w_buf[pl.ds(i * 16, 16)] = zeros

        for r in range(SR // KA + (1 if SR % KA else 0)):
            n = min(KA, SR - r * KA)
            pltpu.sync_copy(w_buf.at[pl.ds(0, n)],
                            den_sp.at[pl.ds(sid * SR + r * KA, n)])
        plsc.subcore_barrier()

        @pl.loop(0, NPASSA)
        def _(j):
            off = base + j * KA
            pltpu.sync_copy(srcE.at[pl.ds(off, KA)], s_buf)
            for r in range(KA // QB):
                pltpu.sync_copy(dstE.at[pl.ds(off + r * QB, QB)], d_buf.at[r])

            @pl.loop(0, KA // 16)
            def _(i):
                s = s_buf[pl.ds(i * 16, 16)]
                d = d_buf[i * 16 // QB, pl.ds((i * 16) % QB, 16)]
                a = plsc.load_gather(as_t, [s]) + plsc.load_gather(ad_t, [d])
                w_buf[pl.ds(i * 16, 16)] = jnp.exp(jnp.maximum(a, a * 0.2))

            pltpu.sync_copy(w_buf, w2.at[pl.ds(head * EP + off, KA)])
            for r in range(KA // QB):
                pltpu.sync_copy(w_buf.at[pl.ds(r * QB, QB)],
                                den_sp.at[d_buf.at[r]], add=True)

        plsc.subcore_barrier()
        pltpu.sync_copy(
            den_sp.at[pl.ds(sid * SR, SR)],
            denp.at[pl.ds((cid * HEADS + head) * NP + sid * SR, SR)])
        plsc.subcore_barrier()


def _sc_weights(asrc, adst, srcE, dstE):
    mesh = plsc.VectorSubcoreMesh(core_axis_name="c", subcore_axis_name="s")
    f = pl.kernel(
        _sc_weights_body,
        out_type=[
            jax.ShapeDtypeStruct((HEADS * EP,), jnp.float32),
            jax.ShapeDtypeStruct((2 * HEADS * NP,), jnp.float32),
        ],
        mesh=mesh,
        compiler_params=pltpu.CompilerParams(needs_layout_passes=False),
        scratch_types=[
            pltpu.VMEM((KA,), jnp.int32),
            pltpu.VMEM((KA // QB, QB), jnp.int32),
            pltpu.VMEM((KA,), jnp.float32),
            pltpu.VMEM((NP,), jnp.float32),
            pltpu.VMEM((NP,), jnp.float32),
            pltpu.VMEM_SHARED((NP,), jnp.float32),
        ],
    )
    return f(asrc, adst, srcE, dstE)


def _sc_body(C, G, h2d, w2, srcE, dstE, num, bufs, accum):
    HG = RW // C
    cid = lax.axis_index("c")
    sid = lax.axis_index("s")
    base = sid * TE
    zeros = jnp.zeros((16,), jnp.float32)

    def fetch(b, g, ha, hb, off):
        src_buf, srh_buf, dst_buf, wa_buf, wb_buf, rows_buf, sem = b
        pltpu.sync_copy(srcE.at[pl.ds(off, K)], src_buf)
        for r in range(K // QB):
            pltpu.sync_copy(dstE.at[pl.ds(off + r * QB, QB)], dst_buf.at[r])
        pltpu.sync_copy(w2.at[pl.ds(ha * EP + off, K)], wa_buf)
        if HG == 2:
            pltpu.sync_copy(w2.at[pl.ds(hb * EP + off, K)], wb_buf)

        @pl.loop(0, K // 16)
        def _(i):
            srh_buf[i * 16 // QB, pl.ds((i * 16) % QB, 16)] = (
                src_buf[pl.ds(i * 16, 16)] + g * NP)

        return [pltpu.async_copy(h2d.at[srh_buf.at[r]],
                                 rows_buf.at[pl.ds(r * QB, QB)], sem)
                for r in range(K // QB)]

    def process(b, cps):
        src_buf, srh_buf, dst_buf, wa_buf, wb_buf, rows_buf, sem = b
        for cp in cps:
            cp.wait()

        @pl.loop(0, K // 16)
        def _(i):
            wav = wa_buf[pl.ds(i * 16, 16)]
            wbv = wb_buf[pl.ds(i * 16, 16)] if HG == 2 else wav
            for l in range(16):
                e = i * 16 + l
                wa = wav[l]
                wb = wbv[l]
                for c in range(RW // 16):
                    w = wa if c < (RW // 32) else wb
                    rows_buf[e, pl.ds(c * 16, 16)] = (
                        rows_buf[e, pl.ds(c * 16, 16)] * w)

        for r in range(K // QB):
            pltpu.sync_copy(rows_buf.at[pl.ds(r * QB, QB)],
                            accum.at[dst_buf.at[r]], add=True)

    for gg in range(G // 2):
        g = cid * (G // 2) + gg
        ha = g * HG
        hb = g * HG + (HG - 1)
        ba, bb = bufs
        rows_a = ba[5]

        @pl.loop(0, K)
        def _(e):
            for c in range(RW // 16):
                rows_a[e, pl.ds(c * 16, 16)] = zeros

        for r in range(SR // K + (1 if SR % K else 0)):
            n = min(K, SR - r * K)
            pltpu.sync_copy(rows_a.at[pl.ds(0, n)],
                            accum.at[pl.ds(sid * SR + r * K, n)])
        plsc.subcore_barrier()

        cp_a = fetch(ba, g, ha, hb, base)

        @pl.loop(0, NPASS // 2 - 1)
        def _(j):
            off = base + j * (2 * K)
            cp_b = fetch(bb, g, ha, hb, off + K)
            process(ba, cp_a)
            fetch(ba, g, ha, hb, off + 2 * K)
            process(bb, cp_b)

        off = base + (NPASS // 2 - 1) * (2 * K)
        cp_b = fetch(bb, g, ha, hb, off + K)
        process(ba, cp_a)
        process(bb, cp_b)

        plsc.subcore_barrier()
        pltpu.sync_copy(accum.at[pl.ds(sid * SR, SR)],
                        num.at[g, pl.ds(sid * SR, SR)])
        plsc.subcore_barrier()


def _sc_aggregate(h2d, w2, srcE, dstE, C):
    G = h2d.shape[0] // NP
    mesh = plsc.VectorSubcoreMesh(core_axis_name="c", subcore_axis_name="s")
    bufset = [
        pltpu.VMEM((K,), jnp.int32),
        pltpu.VMEM((K // QB, QB), jnp.int32),
        pltpu.VMEM((K // QB, QB), jnp.int32),
        pltpu.VMEM((K,), jnp.float32),
        pltpu.VMEM((K,), jnp.float32),
        pltpu.VMEM((K, RW), jnp.float32),
        pltpu.SemaphoreType.DMA,
    ]
    f = pl.kernel(
        functools.partial(_sc_body, C, G),
        out_type=jax.ShapeDtypeStruct((G, NP, RW), jnp.float32),
        mesh=mesh,
        compiler_params=pltpu.CompilerParams(needs_layout_passes=False),
        scratch_types=[
            (tuple(bufset), tuple(bufset)),
            pltpu.VMEM_SHARED((NP, RW), jnp.float32),
        ],
    )
    return f(h2d, w2, srcE, dstE)


def _layer(xp, W, att_s, att_d, b, srcE, dstE, C, relu):
    IN = xp.shape[1]
    G = HEADS * C // RW
    HG = RW // C
    Wg = W.reshape(IN, G, RW).transpose(1, 0, 2)
    h, asrc, adst = _project(xp, Wg, att_s.reshape(G, 1, RW),
                             att_d.reshape(G, 1, RW), C)
    w2, denp = _sc_weights(asrc.reshape(HEADS * NP), adst.reshape(HEADS * NP),
                           srcE, dstE)
    num = _sc_aggregate(h.reshape(G * NP, RW), w2, srcE, dstE, C)
    return _epilogue(num, denp.reshape(2, G, HG, NP), b, relu)


def kernel(x, edge_index, edge_attr, W1, att1_src, att1_dst, b1,
           W2, att2_src, att2_dst, b2):
    del edge_attr
    xp = jnp.pad(x, ((0, NP - N), (0, 0)))
    loops = jnp.arange(N, dtype=jnp.int32)
    pad = EP - E - N
    padv = N + (jnp.arange(pad, dtype=jnp.int32) % (NP - N))
    srcE = jnp.concatenate([edge_index[0], loops, padv])
    dstE = jnp.concatenate([edge_index[1], loops, padv])

    h1 = _layer(xp, W1, att1_src, att1_dst, b1, srcE, dstE, 128, True)
    out = _layer(h1, W2, att2_src, att2_dst, b2, srcE, dstE, 64, False)
    return out[:N]


# speedup vs baseline: 17.5466x; 1.0514x over previous
import functools

import jax
import jax.numpy as jnp
from jax import lax
from jax.experimental import pallas as pl
from jax.experimental.pallas import tpu as pltpu
from jax.experimental.pallas import tpu_sc as plsc

HEADS = 8
N = 10000
NP = 10240
E = 320000
EP = 332800
K = 160
TE = EP // 16
NPASS = TE // K
SR = NP // 16
KA = 400
EA = EP // 32
NPASSA = EA // KA
QB = 80


RW = 128
CB = 64


def _project_body(x_ref, w_ref, as_ref, ad_ref, h_ref, asrc_ref, adst_ref,
                  *, C):
    HG = RW // C
    h = jnp.dot(x_ref[...], w_ref[0], preferred_element_type=jnp.float32)
    h_ref[0] = h
    for hg in range(HG):
        hs = h[:, hg * C:(hg + 1) * C]
        asrc_ref[0, hg] = (hs * as_ref[0, 0, hg * C:(hg + 1) * C]).sum(axis=-1)
        adst_ref[0, hg] = (hs * ad_ref[0, 0, hg * C:(hg + 1) * C]).sum(axis=-1)


def _project(xp, Wg, att_s, att_d, C, interpret=False):
    IN = xp.shape[1]
    G = Wg.shape[0]
    HG = RW // C
    BN = 512
    grid = (G, NP // BN)
    return pl.pallas_call(
        functools.partial(_project_body, C=C),
        grid=grid,
        in_specs=[
            pl.BlockSpec((BN, IN), lambda g, nb: (nb, 0)),
            pl.BlockSpec((1, IN, RW), lambda g, nb: (g, 0, 0)),
            pl.BlockSpec((1, 1, RW), lambda g, nb: (g, 0, 0)),
            pl.BlockSpec((1, 1, RW), lambda g, nb: (g, 0, 0)),
        ],
        out_specs=[
            pl.BlockSpec((1, BN, RW), lambda g, nb: (g, nb, 0)),
            pl.BlockSpec((1, HG, BN), lambda g, nb: (g, 0, nb)),
            pl.BlockSpec((1, HG, BN), lambda g, nb: (g, 0, nb)),
        ],
        out_shape=[
            jax.ShapeDtypeStruct((G, NP, RW), jnp.float32),
            jax.ShapeDtypeStruct((G, HG, NP), jnp.float32),
            jax.ShapeDtypeStruct((G, HG, NP), jnp.float32),
        ],
        interpret=interpret,
    )(xp, Wg, att_s, att_d)


def _epilogue_body(num_ref, den_ref, b_ref, o_ref, *, C, relu, bn):
    HG = RW // C
    G = num_ref.shape[0]
    v = num_ref[...]
    d = den_ref[0] + den_ref[1]
    m = jnp.zeros(v.shape[1:2] + (C,), jnp.float32)
    for hg in range(HG):
        m = m + (v[:, :, hg * C:(hg + 1) * C]
                 / d[:, hg, :, None]).sum(axis=0)
    m = m * (1.0 / HEADS) + b_ref[...][None, :]
    if relu:
        m = jnp.maximum(m, 0.0)
    nb = pl.program_id(0)
    row = nb * bn + lax.broadcasted_iota(jnp.int32, m.shape, 0)
    o_ref[...] = jnp.where(row < N, m, 0.0)


def _epilogue(num, den, b, relu, interpret=False):
    G = num.shape[0]
    HG = HEADS // G
    C = RW // HG
    BN = 512
    return pl.pallas_call(
        functools.partial(_epilogue_body, C=C, relu=relu, bn=BN),
        grid=(NP // BN,),
        in_specs=[
            pl.BlockSpec((G, BN, RW), lambda nb: (0, nb, 0)),
            pl.BlockSpec((2, G, HG, BN), lambda nb: (0, 0, 0, nb)),
            pl.BlockSpec((C,), lambda nb: (0,)),
        ],
        out_specs=pl.BlockSpec((BN, C), lambda nb: (nb, 0)),
        out_shape=jax.ShapeDtypeStruct((NP, C), jnp.float32),
        interpret=interpret,
    )(num, den, b)


def _sc_weights_body(asrc, adst, srcE, dstE, w2, denp,
                     s_buf, d_buf, w_buf, as_t, ad_t, den_sp):
    cid = lax.axis_index("c")
    sid = lax.axis_index("s")
    base = (cid * 16 + sid) * EA
    zeros = jnp.zeros((16,), jnp.float32)

    for head in range(HEADS):
        pltpu.sync_copy(asrc.at[pl.ds(head * NP, NP)], as_t)
        pltpu.sync_copy(adst.at[pl.ds(head * NP, NP)], ad_t)

        @pl.loop(0, KA // 16)
        def _(i):
            w_buf[pl.ds(i * 16, 16)] = zeros

        for r in range(SR // KA + (1 if SR % KA else 0)):
            n = min(KA, SR - r * KA)
            pltpu.sync_copy(w_buf.at[pl.ds(0, n)],
                            den_sp.at[pl.ds(sid * SR + r * KA, n)])
        plsc.subcore_barrier()

        @pl.loop(0, NPASSA)
        def _(j):
            off = base + j * KA
            pltpu.sync_copy(srcE.at[pl.ds(off, KA)], s_buf)
            for r in range(KA // QB):
                pltpu.sync_copy(dstE.at[pl.ds(off + r * QB, QB)], d_buf.at[r])

            @pl.loop(0, KA // 16)
            def _(i):
                s = s_buf[pl.ds(i * 16, 16)]
                d = d_buf[i * 16 // QB, pl.ds((i * 16) % QB, 16)]
                a = plsc.load_gather(as_t, [s]) + plsc.load_gather(ad_t, [d])
                w_buf[pl.ds(i * 16, 16)] = jnp.exp(jnp.maximum(a, a * 0.2))

            pltpu.sync_copy(w_buf, w2.at[pl.ds(head * EP + off, KA)])
            for r in range(KA // QB):
                pltpu.sync_copy(w_buf.at[pl.ds(r * QB, QB)],
                                den_sp.at[d_buf.at[r]], add=True)

        plsc.subcore_barrier()
        pltpu.sync_copy(
            den_sp.at[pl.ds(sid * SR, SR)],
            denp.at[pl.ds((cid * HEADS + head) * NP + sid * SR, SR)])
        plsc.subcore_barrier()


def _sc_weights(asrc, adst, srcE, dstE):
    mesh = plsc.VectorSubcoreMesh(core_axis_name="c", subcore_axis_name="s")
    f = pl.kernel(
        _sc_weights_body,
        out_type=[
            jax.ShapeDtypeStruct((HEADS * EP,), jnp.float32),
            jax.ShapeDtypeStruct((2 * HEADS * NP,), jnp.float32),
        ],
        mesh=mesh,
        compiler_params=pltpu.CompilerParams(needs_layout_passes=False),
        scratch_types=[
            pltpu.VMEM((KA,), jnp.int32),
            pltpu.VMEM((KA // QB, QB), jnp.int32),
            pltpu.VMEM((KA,), jnp.float32),
            pltpu.VMEM((NP,), jnp.float32),
            pltpu.VMEM((NP,), jnp.float32),
            pltpu.VMEM_SHARED((NP,), jnp.float32),
        ],
    )
    return f(asrc, adst, srcE, dstE)


def _sc_body(C, G, h2d, w2, srcE, dstE, num, bufs, accum):
    HG = RW // C
    cid = lax.axis_index("c")
    sid = lax.axis_index("s")
    base = sid * TE
    zeros = jnp.zeros((16,), jnp.float32)

    def fetch(b, g, ha, hb, off):
        src_buf, srh_buf, dst_buf, wa_buf, wb_buf, rows_buf, sem = b
        pltpu.sync_copy(srcE.at[pl.ds(off, K)], src_buf)
        for r in range(K // QB):
            pltpu.sync_copy(dstE.at[pl.ds(off + r * QB, QB)], dst_buf.at[r])
        pltpu.sync_copy(w2.at[pl.ds(ha * EP + off, K)], wa_buf)
        if HG == 2:
            pltpu.sync_copy(w2.at[pl.ds(hb * EP + off, K)], wb_buf)

        @pl.loop(0, K // 16)
        def _(i):
            srh_buf[i * 16 // QB, pl.ds((i * 16) % QB, 16)] = (
                src_buf[pl.ds(i * 16, 16)] + g * NP)

        return [pltpu.async_copy(h2d.at[srh_buf.at[r]],
                                 rows_buf.at[pl.ds(r * QB, QB)], sem)
                for r in range(K // QB)]

    def process(b, cps):
        src_buf, srh_buf, dst_buf, wa_buf, wb_buf, rows_buf, sem = b
        for cp in cps:
            cp.wait()

        @pl.loop(0, K // 16)
        def _(i):
            wav = wa_buf[pl.ds(i * 16, 16)]
            wbv = wb_buf[pl.ds(i * 16, 16)] if HG == 2 else wav
            for l in range(16):
                e = i * 16 + l
                wa = wav[l]
                wb = wbv[l]
                for c in range(RW // 16):
                    w = wa if c < (RW // 32) else wb
                    rows_buf[e, pl.ds(c * 16, 16)] = (
                        rows_buf[e, pl.ds(c * 16, 16)] * w)

        for r in range(K // QB):
            pltpu.sync_copy(rows_buf.at[pl.ds(r * QB, QB)],
                            accum.at[dst_buf.at[r]], add=True)

    for gg in range(G // 2):
        g = cid * (G // 2) + gg
        ha = g * HG
        hb = g * HG + (HG - 1)
        ba, bb = bufs
        rows_a = ba[5]

        @pl.loop(0, K)
        def _(e):
            for c in range(RW // 16):
                rows_a[e, pl.ds(c * 16, 16)] = zeros

        for r in range(SR // K + (1 if SR % K else 0)):
            n = min(K, SR - r * K)
            pltpu.sync_copy(rows_a.at[pl.ds(0, n)],
                            accum.at[pl.ds(sid * SR + r * K, n)])
        plsc.subcore_barrier()

        cp_a = fetch(ba, g, ha, hb, base)

        @pl.loop(0, NPASS // 2 - 1)
        def _(j):
            off = base + j * (2 * K)
            cp_b = fetch(bb, g, ha, hb, off + K)
            process(ba, cp_a)
            fetch(ba, g, ha, hb, off + 2 * K)
            process(bb, cp_b)

        off = base + (NPASS // 2 - 1) * (2 * K)
        cp_b = fetch(bb, g, ha, hb, off + K)
        process(ba, cp_a)
        process(bb, cp_b)

        plsc.subcore_barrier()
        pltpu.sync_copy(accum.at[pl.ds(sid * SR, SR)],
                        num.at[g, pl.ds(sid * SR, SR)])
        plsc.subcore_barrier()


def _sc_aggregate(h2d, w2, srcE, dstE, C):
    G = h2d.shape[0] // NP
    mesh = plsc.VectorSubcoreMesh(core_axis_name="c", subcore_axis_name="s")
    bufset = [
        pltpu.VMEM((K,), jnp.int32),
        pltpu.VMEM((K // QB, QB), jnp.int32),
        pltpu.VMEM((K // QB, QB), jnp.int32),
        pltpu.VMEM((K,), jnp.float32),
        pltpu.VMEM((K,), jnp.float32),
        pltpu.VMEM((K, RW), jnp.float32),
        pltpu.SemaphoreType.DMA,
    ]
    f = pl.kernel(
        functools.partial(_sc_body, C, G),
        out_type=jax.ShapeDtypeStruct((G, NP, RW), jnp.float32),
        mesh=mesh,
        compiler_params=pltpu.CompilerParams(needs_layout_passes=False),
        scratch_types=[
            (tuple(bufset), tuple(bufset)),
            pltpu.VMEM_SHARED((NP, RW), jnp.float32),
        ],
    )
    return f(h2d, w2, srcE, dstE)


def _layer(xp, W, att_s, att_d, b, srcE, dstE, C, relu):
    IN = xp.shape[1]
    G = HEADS * C // RW
    HG = RW // C
    Wg = W.reshape(IN, G, RW).transpose(1, 0, 2)
    h, asrc, adst = _project(xp, Wg, att_s.reshape(G, 1, RW),
                             att_d.reshape(G, 1, RW), C)
    w2, denp = _sc_weights(asrc.reshape(HEADS * NP), adst.reshape(HEADS * NP),
                           srcE, dstE)
    num = _sc_aggregate(h.reshape(G * NP, RW), w2, srcE, dstE, C)
    return _epilogue(num, denp.reshape(2, G, HG, NP), b, relu)


def kernel(x, edge_index, edge_attr, W1, att1_src, att1_dst, b1,
           W2, att2_src, att2_dst, b2):
    del edge_attr
    xp = jnp.pad(x, ((0, NP - N), (0, 0)))
    loops = jnp.arange(N, dtype=jnp.int32)
    pad = EP - E - N
    padv = N + (jnp.arange(pad, dtype=jnp.int32) % (NP - N))
    srcE = jnp.concatenate([edge_index[0], loops, padv])
    dstE = jnp.concatenate([edge_index[1], loops, padv])

    h1 = _layer(xp, W1, att1_src, att1_dst, b1, srcE, dstE, 128, True)
    out = _layer(h1, W2, att2_src, att2_dst, b2, srcE, dstE, 64, False)
    return out[:N]
